# baseline (speedup 1.0000x reference)
import dataclasses
import functools

import jax
import jax.numpy as jnp
from jax import lax
from jax.experimental import pallas as pl
from jax.experimental.pallas import tpu as pltpu
from jax.experimental.pallas import tpu_sc as plsc

EMBED = 64
LINE = 2 * EMBED
NLINES = 500000
LANES = 16
NCORE = 2
NSUB = 16
NWORK = NCORE * NSUB
BATCH = 16384
BPW = BATCH // NWORK
PCHUNK = 128
NCH = BPW // PCHUNK
GROUP = LANES
KCH = EMBED // LANES

_cp = pltpu.CompilerParams(use_tc_tiling_on_sc=False)
if "needs_layout_passes" in pltpu.CompilerParams.__dataclass_fields__:
    _cp = dataclasses.replace(_cp, needs_layout_passes=False)


@functools.partial(
    pl.kernel,
    compiler_params=_cp,
    out_type=jax.ShapeDtypeStruct((BATCH,), jnp.float32),
    mesh=plsc.VectorSubcoreMesh(core_axis_name="c", subcore_axis_name="s"),
    scratch_types=[
        pltpu.VMEM((BPW,), jnp.int32),
        pltpu.VMEM((BPW,), jnp.int32),
        pltpu.VMEM((BPW,), jnp.int32),
        pltpu.VMEM((BPW,), jnp.int32),
        pltpu.VMEM((PCHUNK, LINE), jnp.float32),
        pltpu.VMEM((PCHUNK, LINE), jnp.float32),
        pltpu.VMEM((GROUP, LANES), jnp.float32),
        pltpu.VMEM((BPW,), jnp.float32),
        pltpu.SemaphoreType.DMA,
    ],
)
def _sgns_sc(c_hbm, w_hbm, cpack_hbm, wpack_hbm, out_hbm,
             cline, wline, chalf, whalf, cbuf, wbuf, accbuf, outv, sem):
    wid = lax.axis_index("s") * NCORE + lax.axis_index("c")
    base = wid * BPW

    pltpu.sync_copy(c_hbm.at[pl.ds(base, BPW)], cline)
    pltpu.sync_copy(w_hbm.at[pl.ds(base, BPW)], wline)
    ione = jnp.full((LANES,), 1, jnp.int32)
    for v in range(BPW // LANES):
        sl = pl.ds(v * LANES, LANES)
        cv = cline[sl]
        wv = wline[sl]
        chalf[sl] = cv & ione
        whalf[sl] = wv & ione
        cline[sl] = cv >> 1
        wline[sl] = wv >> 1

    row_iota = lax.iota(jnp.int32, LANES)
    fone = jnp.full((LANES,), 1.0, jnp.float32)
    izero = jnp.zeros((LANES,), jnp.int32)

    @pl.loop(0, NCH)
    def _(ch):
        p0 = ch * PCHUNK
        cg = pltpu.async_copy(
            cpack_hbm.at[cline.at[pl.ds(p0, PCHUNK)]], cbuf, sem)
        wg = pltpu.async_copy(
            wpack_hbm.at[wline.at[pl.ds(p0, PCHUNK)]], wbuf, sem)
        cg.wait()
        wg.wait()

        for g in range(PCHUNK // GROUP):
            for r in range(GROUP):
                p = g * GROUP + r
                psplat = jnp.full((LANES,), p0 + p, jnp.int32)
                cmask = plsc.load_gather(chalf, [psplat]) > izero
                wmask = plsc.load_gather(whalf, [psplat]) > izero
                acc = None
                for k in range(KCH):
                    clo = cbuf[p, pl.ds(k * LANES, LANES)]
                    chi = cbuf[p, pl.ds(EMBED + k * LANES, LANES)]
                    wlo = wbuf[p, pl.ds(k * LANES, LANES)]
                    whi = wbuf[p, pl.ds(EMBED + k * LANES, LANES)]
                    cv = jnp.where(cmask, chi, clo)
                    wv = jnp.where(wmask, whi, wlo)
                    prod = cv * wv
                    acc = prod if acc is None else acc + prod
                accbuf[r, :] = acc
            tot = None
            for j in range(LANES):
                col = plsc.load_gather(
                    accbuf, [row_iota, jnp.full((LANES,), j, jnp.int32)])
                tot = col if tot is None else tot + col
            outv[pl.ds(p0 + g * GROUP, GROUP)] = fone / (fone + jnp.exp(-tot))

    pltpu.sync_copy(outv, out_hbm.at[pl.ds(base, BPW)])


def kernel(c, w, c_table, w_table):
    cpack, wpack = jax.lax.optimization_barrier(
        (c_table.reshape(NLINES, LINE), w_table.reshape(NLINES, LINE)))
    return _sgns_sc(c, w, cpack, wpack)
